# Initial kernel scaffold; baseline (speedup 1.0000x reference)
import jax
import jax.numpy as jnp
from jax.experimental import pallas as pl


def kernel(emission):
    raise NotImplementedError("write your pallas kernel here")


# speedup vs baseline: 3.8405x; 3.8405x over previous
import functools

import jax
import jax.numpy as jnp
from jax import lax
from jax.experimental import pallas as pl
from jax.experimental.pallas import tpu as pltpu
from jax.experimental.pallas import tpu_sc as plsc


_LANES = 16

_GATHER_DNUMS = lax.GatherDimensionNumbers(
    offset_dims=(), collapsed_slice_dims=(0,), start_index_map=(0,)
)


def _vgather(v, idx):
    return lax.gather(
        v,
        idx[:, None],
        _GATHER_DNUMS,
        slice_sizes=(1,),
        mode=lax.GatherScatterMode.PROMISE_IN_BOUNDS,
    )


def _argmax_body(x_ref, out_ref):
    x = x_ref[...]
    m = jnp.max(x, axis=-1, keepdims=True)
    ii = lax.broadcasted_iota(jnp.int32, x.shape, 2)
    C = x.shape[-1]
    out_ref[...] = jnp.min(jnp.where(x >= m, ii, C), axis=-1).astype(jnp.int32)


def _dedup_body(T, blank, idx_hbm, tok_hbm, len_hbm, row_v, out_v, len_v):
    wid = lax.axis_index("s") * 2 + lax.axis_index("c")
    pltpu.sync_copy(idx_hbm.at[wid], row_v)

    nchunks = T // _LANES
    lane = lax.iota(jnp.int32, _LANES)
    shift_idx = jnp.maximum(lane - 1, 0)
    last_idx = jnp.minimum(lane + _LANES, _LANES - 1) * 0 + (_LANES - 1)

    def fill(i, _):
        out_v[pl.ds(i * _LANES, _LANES)] = jnp.zeros((_LANES,), jnp.int32) - 1
        return 0

    lax.fori_loop(0, nchunks, fill, 0)

    def body(i, carry):
        total, prev_last = carry
        base = i * _LANES
        v = row_v[pl.ds(base, _LANES)]
        shifted = _vgather(v, shift_idx)
        prev = jnp.where(lane == 0, prev_last, shifted)
        keep = (v != prev) & (v != blank)
        plsc.store_compressed(out_v.at[pl.ds(total, _LANES)], v, mask=keep)
        new_last = _vgather(v, last_idx)
        return total + jnp.sum(keep.astype(jnp.int32)), new_last

    init = (jnp.int32(0), jnp.zeros((_LANES,), jnp.int32) - 1)
    total, _ = lax.fori_loop(0, nchunks, body, init)

    pltpu.sync_copy(out_v, tok_hbm.at[wid])
    len_v[...] = jnp.zeros((_LANES,), jnp.int32) + total
    pltpu.sync_copy(len_v, len_hbm.at[wid])


def kernel(emission):
    T, N, C = emission.shape
    blank = C - 1
    TB = 128

    idx_tn = pl.pallas_call(
        _argmax_body,
        grid=(T // TB,),
        in_specs=[pl.BlockSpec((TB, N, C), lambda i: (i, 0, 0))],
        out_specs=pl.BlockSpec((TB, N), lambda i: (i, 0)),
        out_shape=jax.ShapeDtypeStruct((T, N), jnp.int32),
    )(emission)
    idx = idx_tn.T

    mesh = plsc.VectorSubcoreMesh(core_axis_name="c", subcore_axis_name="s")
    dedup = functools.partial(
        pl.kernel,
        mesh=mesh,
        out_type=[
            jax.ShapeDtypeStruct((N, T), jnp.int32),
            jax.ShapeDtypeStruct((N, _LANES), jnp.int32),
        ],
        scratch_types=[
            pltpu.VMEM((T,), jnp.int32),
            pltpu.VMEM((T,), jnp.int32),
            pltpu.VMEM((_LANES,), jnp.int32),
        ],
        compiler_params=pltpu.CompilerParams(needs_layout_passes=False),
    )(functools.partial(_dedup_body, T, blank))

    tokens, len_pad = dedup(idx)
    lengths = len_pad[:, 0]
    return tokens, lengths
